# baseline (speedup 1.0000x reference)
import functools

import jax
import jax.numpy as jnp
from jax import lax
from jax.experimental import pallas as pl
from jax.experimental.pallas import tpu as pltpu
from jax.experimental.pallas import tpu_sc as plsc

N = 10000
F = 128
G = 16
C = 10

NC = 2
NS = 16
NW = NC * NS
CH = 128

NPAD = 10240
RPS = NPAD // NS


def _edge_setup(edge_index):
    e = edge_index.shape[1]
    etot = e + N
    kch = ((etot + NW - 1) // NW + CH - 1) // CH
    epad = NW * kch * CH
    loop = jnp.arange(N, dtype=jnp.int32)
    pad = jnp.full((epad - etot,), N, dtype=jnp.int32)
    srcs = jnp.concatenate([edge_index[0], loop, pad]).reshape(NW, kch, CH)
    dsts = jnp.concatenate([edge_index[1], loop, pad]).reshape(NW, kch, CH)
    return srcs, dsts, kch


def _make_deg_kernel(kch):
    mesh = plsc.VectorSubcoreMesh(core_axis_name="c", subcore_axis_name="s")

    @functools.partial(
        pl.kernel,
        out_type=jax.ShapeDtypeStruct((NC, NPAD), jnp.float32),
        mesh=mesh,
        scratch_types=[
            pltpu.VMEM((kch, CH), jnp.int32),
            pltpu.VMEM((CH,), jnp.float32),
            pltpu.VMEM((RPS,), jnp.float32),
            pltpu.VMEM_SHARED((NPAD,), jnp.float32),
            pltpu.SemaphoreType.DMA,
        ],
        compiler_params=pltpu.CompilerParams(use_tc_tiling_on_sc=False),
    )
    def deg_kernel(dsts_hbm, z_hbm, out_hbm, dst_v, ones_v, stage_v, deg_sh,
                   sem):
        cid = lax.axis_index("c")
        sid = lax.axis_index("s")
        wid = sid * NC + cid
        pltpu.sync_copy(dsts_hbm.at[wid], dst_v)
        for i in range(CH // 16):
            ones_v[pl.ds(i * 16, 16)] = jnp.ones((16,), jnp.float32)
        pltpu.sync_copy(z_hbm.at[pl.ds(sid * RPS, RPS)],
                        deg_sh.at[pl.ds(sid * RPS, RPS)])
        plsc.subcore_barrier()

        descs = [pltpu.async_copy(ones_v, deg_sh.at[dst_v.at[j]], sem, add=True)
                 for j in range(kch)]
        for d in descs:
            d.wait()
        plsc.subcore_barrier()
        pltpu.sync_copy(deg_sh.at[pl.ds(sid * RPS, RPS)], stage_v)
        pltpu.sync_copy(stage_v, out_hbm.at[cid, pl.ds(sid * RPS, RPS)])

    return deg_kernel


NBUF = 8
PREF = 4


def _make_agg_kernel(kch, dout):
    mesh = plsc.VectorSubcoreMesh(core_axis_name="c", subcore_axis_name="s")
    assert kch >= NBUF

    @functools.partial(
        pl.kernel,
        out_type=jax.ShapeDtypeStruct((NC, NPAD, dout), jnp.float32),
        mesh=mesh,
        scratch_types=(
            [pltpu.VMEM((kch, CH), jnp.int32),
             pltpu.VMEM((kch, CH), jnp.int32),
             pltpu.VMEM((RPS, dout), jnp.float32),
             pltpu.VMEM_SHARED((NPAD, dout), jnp.float32)]
            + [pltpu.VMEM((CH, dout), jnp.float32) for _ in range(NBUF)]
            + [pltpu.SemaphoreType.DMA for _ in range(2 * NBUF)]
        ),
        compiler_params=pltpu.CompilerParams(use_tc_tiling_on_sc=False),
    )
    def agg_kernel(ms_hbm, srcs_hbm, dsts_hbm, z_hbm, out_hbm,
                   src_v, dst_v, stage_v, agg_sh, *bufs_and_sems):
        vals = bufs_and_sems[:NBUF]
        gsem = bufs_and_sems[NBUF:2 * NBUF]
        ssem = bufs_and_sems[2 * NBUF:]
        cid = lax.axis_index("c")
        sid = lax.axis_index("s")
        wid = sid * NC + cid
        pltpu.sync_copy(srcs_hbm.at[wid], src_v)
        pltpu.sync_copy(dsts_hbm.at[wid], dst_v)
        pltpu.sync_copy(z_hbm.at[pl.ds(sid * RPS, RPS)],
                        agg_sh.at[pl.ds(sid * RPS, RPS)])
        plsc.subcore_barrier()

        gd = [None] * kch
        sd = [None] * kch
        for j in range(PREF):
            gd[j] = pltpu.async_copy(ms_hbm.at[src_v.at[j]], vals[j % NBUF],
                                     gsem[j % NBUF])
        for j in range(kch):
            jp = j + PREF
            if jp < kch:
                bp = jp % NBUF
                if jp >= NBUF:
                    sd[jp - NBUF].wait()
                gd[jp] = pltpu.async_copy(ms_hbm.at[src_v.at[jp]], vals[bp],
                                          gsem[bp])
            b = j % NBUF
            gd[j].wait()
            sd[j] = pltpu.async_copy(vals[b], agg_sh.at[dst_v.at[j]], ssem[b],
                                     add=True)
        for j in range(kch - NBUF, kch):
            sd[j].wait()
        plsc.subcore_barrier()
        pltpu.sync_copy(agg_sh.at[pl.ds(sid * RPS, RPS)], stage_v)
        pltpu.sync_copy(stage_v, out_hbm.at[cid, pl.ds(sid * RPS, RPS)])

    return agg_kernel


def _prep1_body(xp_ref, w1_ref, degp_ref, dinv_ref, ms1_ref):
    deg = degp_ref[0] + degp_ref[1]
    dinv = lax.rsqrt(jnp.maximum(deg, 1e-12))
    m1 = jnp.dot(xp_ref[...], w1_ref[...], preferred_element_type=jnp.float32)
    dinv_ref[...] = dinv
    ms1_ref[...] = dinv[:, None] * m1


def _mid_body(agg_ref, dinv_ref, b1_ref, w2_ref, ms2_ref):
    dinv = dinv_ref[...]
    h1 = jnp.maximum(dinv[:, None] * (agg_ref[0] + agg_ref[1]) + b1_ref[...], 0.0)
    rowmask = lax.broadcasted_iota(jnp.int32, (NPAD, 1), 0) < N
    h1 = jnp.where(rowmask, h1, 0.0)
    m2 = jnp.dot(h1, w2_ref[...], preferred_element_type=jnp.float32)
    ms2_ref[...] = dinv[:, None] * m2


def _final_body(agg_ref, dinv_ref, b2_ref, batch_ref, wfc_ref, bfc_ref, out_ref):
    dinv = dinv_ref[...]
    h2 = jnp.maximum(dinv[:, None] * (agg_ref[0] + agg_ref[1]) + b2_ref[...], 0.0)
    gids = lax.broadcasted_iota(jnp.int32, (NPAD, G), 1)
    oh = (batch_ref[...][:, None] == gids).astype(jnp.float32)
    sums = lax.dot_general(oh, h2, (((0,), (0,)), ((), ())),
                           preferred_element_type=jnp.float32)
    cnt = jnp.sum(oh, axis=0)
    pooled = sums / jnp.maximum(cnt, 1.0)[:, None]
    out_ref[...] = jnp.dot(pooled, wfc_ref[...],
                           preferred_element_type=jnp.float32) + bfc_ref[...]


def kernel(x, edge_index, batch, W1, b1, W2, b2, Wfc, bfc):
    srcs, dsts, kch = _edge_setup(edge_index)
    xp = jnp.zeros((NPAD, F), jnp.float32).at[:N].set(x)
    batchp = jnp.concatenate(
        [batch.astype(jnp.int32), jnp.full((NPAD - N,), G, jnp.int32)])

    degp = _make_deg_kernel(kch)(dsts, jnp.zeros((NPAD,), jnp.float32))

    dinv, ms1 = pl.pallas_call(
        _prep1_body,
        out_shape=[jax.ShapeDtypeStruct((NPAD,), jnp.float32),
                   jax.ShapeDtypeStruct((NPAD, 16), jnp.float32)],
    )(xp, W1, degp)

    agg1 = _make_agg_kernel(kch, 16)(ms1, srcs, dsts,
                                     jnp.zeros((NPAD, 16), jnp.float32))

    ms2 = pl.pallas_call(
        _mid_body,
        out_shape=jax.ShapeDtypeStruct((NPAD, 32), jnp.float32),
    )(agg1, dinv, b1, W2)

    agg2 = _make_agg_kernel(kch, 32)(ms2, srcs, dsts,
                                     jnp.zeros((NPAD, 32), jnp.float32))

    out = pl.pallas_call(
        _final_body,
        out_shape=jax.ShapeDtypeStruct((G, C), jnp.float32),
    )(agg2, dinv, b2, batchp, Wfc, bfc)
    return out


# speedup vs baseline: 1.2405x; 1.2405x over previous
import functools

import jax
import jax.numpy as jnp
from jax import lax
from jax.experimental import pallas as pl
from jax.experimental.pallas import tpu as pltpu
from jax.experimental.pallas import tpu_sc as plsc

N = 10000
F = 128
G = 16
C = 10

NC = 2
NS = 16
NW = NC * NS
CH = 128

NPAD = 10240
RPS = NPAD // NS

NBUF = 8
PREF = 4


def _chunks(ew):
    out = []
    off = 0
    while off < ew:
        ln = min(CH, ew - off)
        out.append((off, ln))
        off += ln
    return out


def _make_deg_kernel(ew):
    mesh = plsc.VectorSubcoreMesh(core_axis_name="c", subcore_axis_name="s")
    chunks = _chunks(ew)

    @functools.partial(
        pl.kernel,
        out_type=jax.ShapeDtypeStruct((NC, NPAD), jnp.float32),
        mesh=mesh,
        scratch_types=[
            pltpu.VMEM((ew,), jnp.int32),
            pltpu.VMEM((CH,), jnp.float32),
            pltpu.VMEM((RPS,), jnp.float32),
            pltpu.VMEM_SHARED((NPAD,), jnp.float32),
            pltpu.SemaphoreType.DMA,
        ],
        compiler_params=pltpu.CompilerParams(use_tc_tiling_on_sc=False),
    )
    def deg_kernel(dst_hbm, z_hbm, out_hbm, dst_v, ones_v, stage_v, deg_sh,
                   sem):
        cid = lax.axis_index("c")
        sid = lax.axis_index("s")
        wid = sid * NC + cid
        pltpu.sync_copy(dst_hbm.at[pl.ds(wid * ew, ew)], dst_v)
        for i in range(CH // 16):
            ones_v[pl.ds(i * 16, 16)] = jnp.ones((16,), jnp.float32)
        pltpu.sync_copy(z_hbm.at[pl.ds(sid * RPS, RPS)],
                        deg_sh.at[pl.ds(sid * RPS, RPS)])
        plsc.subcore_barrier()

        descs = [pltpu.async_copy(ones_v.at[pl.ds(0, ln)],
                                  deg_sh.at[dst_v.at[pl.ds(off, ln)]],
                                  sem, add=True)
                 for off, ln in chunks]
        for d in descs:
            d.wait()
        plsc.subcore_barrier()
        pltpu.sync_copy(deg_sh.at[pl.ds(sid * RPS, RPS)], stage_v)
        pltpu.sync_copy(stage_v, out_hbm.at[cid, pl.ds(sid * RPS, RPS)])

    return deg_kernel


def _make_agg_kernel(ew, dout):
    mesh = plsc.VectorSubcoreMesh(core_axis_name="c", subcore_axis_name="s")
    chunks = _chunks(ew)
    kch = len(chunks)
    assert kch >= NBUF

    @functools.partial(
        pl.kernel,
        out_type=jax.ShapeDtypeStruct((NC, NPAD, dout), jnp.float32),
        mesh=mesh,
        scratch_types=(
            [pltpu.VMEM((ew,), jnp.int32),
             pltpu.VMEM((ew,), jnp.int32),
             pltpu.VMEM((RPS, dout), jnp.float32),
             pltpu.VMEM_SHARED((NPAD, dout), jnp.float32)]
            + [pltpu.VMEM((CH, dout), jnp.float32) for _ in range(NBUF)]
            + [pltpu.SemaphoreType.DMA for _ in range(2 * NBUF)]
        ),
        compiler_params=pltpu.CompilerParams(use_tc_tiling_on_sc=False),
    )
    def agg_kernel(ms_hbm, src_hbm, dst_hbm, z_hbm, out_hbm,
                   src_v, dst_v, stage_v, agg_sh, *bufs_and_sems):
        vals = bufs_and_sems[:NBUF]
        gsem = bufs_and_sems[NBUF:2 * NBUF]
        ssem = bufs_and_sems[2 * NBUF:]
        cid = lax.axis_index("c")
        sid = lax.axis_index("s")
        wid = sid * NC + cid
        pltpu.sync_copy(src_hbm.at[pl.ds(wid * ew, ew)], src_v)
        pltpu.sync_copy(dst_hbm.at[pl.ds(wid * ew, ew)], dst_v)
        pltpu.sync_copy(z_hbm.at[pl.ds(sid * RPS, RPS)],
                        agg_sh.at[pl.ds(sid * RPS, RPS)])
        plsc.subcore_barrier()

        def start_gather(j):
            off, ln = chunks[j]
            b = j % NBUF
            return pltpu.async_copy(ms_hbm.at[src_v.at[pl.ds(off, ln)]],
                                    vals[b].at[pl.ds(0, ln)], gsem[b])

        def start_scatter(j):
            off, ln = chunks[j]
            b = j % NBUF
            return pltpu.async_copy(vals[b].at[pl.ds(0, ln)],
                                    agg_sh.at[dst_v.at[pl.ds(off, ln)]],
                                    ssem[b], add=True)

        gd = [None] * kch
        sd = [None] * kch
        for j in range(PREF):
            gd[j] = start_gather(j)
        for j in range(kch):
            jp = j + PREF
            if jp < kch:
                if jp >= NBUF:
                    sd[jp - NBUF].wait()
                gd[jp] = start_gather(jp)
            gd[j].wait()
            sd[j] = start_scatter(j)
        for j in range(kch - NBUF, kch):
            sd[j].wait()
        plsc.subcore_barrier()
        pltpu.sync_copy(agg_sh.at[pl.ds(sid * RPS, RPS)], stage_v)
        pltpu.sync_copy(stage_v, out_hbm.at[cid, pl.ds(sid * RPS, RPS)])

    return agg_kernel


def _prep1_body(x_ref, w1_ref, degp_ref, dinv_ref, ms1_ref):
    deg = degp_ref[0, :N] + degp_ref[1, :N] + 1.0
    dinv = lax.rsqrt(jnp.maximum(deg, 1e-12))
    m1 = jnp.dot(x_ref[...], w1_ref[...], preferred_element_type=jnp.float32)
    dinv_ref[...] = dinv
    ms1_ref[...] = dinv[:, None] * m1


def _mid_body(agg_ref, dinv_ref, ms1_ref, b1_ref, w2_ref, ms2_ref):
    dinv = dinv_ref[...]
    agg = agg_ref[0, :N] + agg_ref[1, :N] + ms1_ref[...]
    h1 = jnp.maximum(dinv[:, None] * agg + b1_ref[...], 0.0)
    m2 = jnp.dot(h1, w2_ref[...], preferred_element_type=jnp.float32)
    ms2_ref[...] = dinv[:, None] * m2


def _final_body(agg_ref, dinv_ref, ms2_ref, b2_ref, batch_ref, wfc_ref,
                bfc_ref, out_ref):
    dinv = dinv_ref[...]
    agg = agg_ref[0, :N] + agg_ref[1, :N] + ms2_ref[...]
    h2 = jnp.maximum(dinv[:, None] * agg + b2_ref[...], 0.0)
    gids = lax.broadcasted_iota(jnp.int32, (N, G), 1)
    oh = (batch_ref[...][:, None] == gids).astype(jnp.float32)
    sums = lax.dot_general(oh, h2, (((0,), (0,)), ((), ())),
                           preferred_element_type=jnp.float32)
    cnt = jnp.sum(oh, axis=0)
    pooled = sums / jnp.maximum(cnt, 1.0)[:, None]
    out_ref[...] = jnp.dot(pooled, wfc_ref[...],
                           preferred_element_type=jnp.float32) + bfc_ref[...]


def kernel(x, edge_index, batch, W1, b1, W2, b2, Wfc, bfc):
    e = edge_index.shape[1]
    assert e % NW == 0
    ew = e // NW
    src = edge_index[0]
    dst = edge_index[1]

    degp = _make_deg_kernel(ew)(dst, jnp.zeros((NPAD,), jnp.float32))

    dinv, ms1 = pl.pallas_call(
        _prep1_body,
        out_shape=[jax.ShapeDtypeStruct((N,), jnp.float32),
                   jax.ShapeDtypeStruct((N, 16), jnp.float32)],
    )(x, W1, degp)

    agg1 = _make_agg_kernel(ew, 16)(ms1, src, dst,
                                    jnp.zeros((NPAD, 16), jnp.float32))

    ms2 = pl.pallas_call(
        _mid_body,
        out_shape=jax.ShapeDtypeStruct((N, 32), jnp.float32),
    )(agg1, dinv, ms1, b1, W2)

    agg2 = _make_agg_kernel(ew, 32)(ms2, src, dst,
                                    jnp.zeros((NPAD, 32), jnp.float32))

    out = pl.pallas_call(
        _final_body,
        out_shape=jax.ShapeDtypeStruct((G, C), jnp.float32),
    )(agg2, dinv, ms2, b2, batch.astype(jnp.int32), Wfc, bfc)
    return out


# speedup vs baseline: 1.2489x; 1.0068x over previous
import functools

import jax
import jax.numpy as jnp
from jax import lax
from jax.experimental import pallas as pl
from jax.experimental.pallas import tpu as pltpu
from jax.experimental.pallas import tpu_sc as plsc

N = 10000
F = 128
G = 16
C = 10

NC = 2
NS = 16
NW = NC * NS
CH = 128

NPAD = 10240
RPS = NPAD // NS

NBUF = 12
PREF = 6


def _chunks(ew):
    out = []
    off = 0
    while off < ew:
        ln = min(CH, ew - off)
        out.append((off, ln))
        off += ln
    return out


def _make_deg_kernel(ew):
    mesh = plsc.VectorSubcoreMesh(core_axis_name="c", subcore_axis_name="s")
    chunks = _chunks(ew)

    @functools.partial(
        pl.kernel,
        out_type=jax.ShapeDtypeStruct((NC, NPAD), jnp.float32),
        mesh=mesh,
        scratch_types=[
            pltpu.VMEM((ew,), jnp.int32),
            pltpu.VMEM((CH,), jnp.float32),
            pltpu.VMEM((RPS,), jnp.float32),
            pltpu.VMEM_SHARED((NPAD,), jnp.float32),
            pltpu.SemaphoreType.DMA,
        ],
        compiler_params=pltpu.CompilerParams(use_tc_tiling_on_sc=False),
    )
    def deg_kernel(dst_hbm, z_hbm, out_hbm, dst_v, ones_v, stage_v, deg_sh,
                   sem):
        cid = lax.axis_index("c")
        sid = lax.axis_index("s")
        wid = sid * NC + cid
        pltpu.sync_copy(dst_hbm.at[pl.ds(wid * ew, ew)], dst_v)
        for i in range(CH // 16):
            ones_v[pl.ds(i * 16, 16)] = jnp.ones((16,), jnp.float32)
        pltpu.sync_copy(z_hbm.at[pl.ds(sid * RPS, RPS)],
                        deg_sh.at[pl.ds(sid * RPS, RPS)])
        plsc.subcore_barrier()

        descs = [pltpu.async_copy(ones_v.at[pl.ds(0, ln)],
                                  deg_sh.at[dst_v.at[pl.ds(off, ln)]],
                                  sem, add=True)
                 for off, ln in chunks]
        for d in descs:
            d.wait()
        plsc.subcore_barrier()
        pltpu.sync_copy(deg_sh.at[pl.ds(sid * RPS, RPS)], stage_v)
        pltpu.sync_copy(stage_v, out_hbm.at[cid, pl.ds(sid * RPS, RPS)])

    return deg_kernel


def _make_agg_kernel(ew, dout):
    mesh = plsc.VectorSubcoreMesh(core_axis_name="c", subcore_axis_name="s")
    chunks = _chunks(ew)
    kch = len(chunks)
    assert kch >= NBUF

    @functools.partial(
        pl.kernel,
        out_type=jax.ShapeDtypeStruct((NC, NPAD, dout), jnp.float32),
        mesh=mesh,
        scratch_types=(
            [pltpu.VMEM((ew,), jnp.int32),
             pltpu.VMEM((ew,), jnp.int32),
             pltpu.VMEM((RPS, dout), jnp.float32),
             pltpu.VMEM_SHARED((NPAD, dout), jnp.float32)]
            + [pltpu.VMEM((CH, dout), jnp.float32) for _ in range(NBUF)]
            + [pltpu.SemaphoreType.DMA for _ in range(2 * NBUF)]
        ),
        compiler_params=pltpu.CompilerParams(use_tc_tiling_on_sc=False),
    )
    def agg_kernel(ms_hbm, src_hbm, dst_hbm, z_hbm, out_hbm,
                   src_v, dst_v, stage_v, agg_sh, *bufs_and_sems):
        vals = bufs_and_sems[:NBUF]
        gsem = bufs_and_sems[NBUF:2 * NBUF]
        ssem = bufs_and_sems[2 * NBUF:]
        cid = lax.axis_index("c")
        sid = lax.axis_index("s")
        wid = sid * NC + cid
        pltpu.sync_copy(src_hbm.at[pl.ds(wid * ew, ew)], src_v)
        pltpu.sync_copy(dst_hbm.at[pl.ds(wid * ew, ew)], dst_v)
        pltpu.sync_copy(z_hbm.at[pl.ds(sid * RPS, RPS)],
                        agg_sh.at[pl.ds(sid * RPS, RPS)])
        plsc.subcore_barrier()

        def start_gather(j):
            off, ln = chunks[j]
            b = j % NBUF
            return pltpu.async_copy(ms_hbm.at[src_v.at[pl.ds(off, ln)]],
                                    vals[b].at[pl.ds(0, ln)], gsem[b])

        def start_scatter(j):
            off, ln = chunks[j]
            b = j % NBUF
            return pltpu.async_copy(vals[b].at[pl.ds(0, ln)],
                                    agg_sh.at[dst_v.at[pl.ds(off, ln)]],
                                    ssem[b], add=True)

        gd = [None] * kch
        sd = [None] * kch
        for j in range(PREF):
            gd[j] = start_gather(j)
        for j in range(kch):
            jp = j + PREF
            if jp < kch:
                if jp >= NBUF:
                    sd[jp - NBUF].wait()
                gd[jp] = start_gather(jp)
            gd[j].wait()
            sd[j] = start_scatter(j)
        for j in range(kch - NBUF, kch):
            sd[j].wait()
        plsc.subcore_barrier()
        pltpu.sync_copy(agg_sh.at[pl.ds(sid * RPS, RPS)], stage_v)
        pltpu.sync_copy(stage_v, out_hbm.at[cid, pl.ds(sid * RPS, RPS)])

    return agg_kernel


def _m1_body(x_ref, w1_ref, m1_ref):
    m1_ref[...] = jnp.dot(x_ref[...], w1_ref[...],
                          preferred_element_type=jnp.float32)


def _prep1_body(m1_ref, degp_ref, dinv_ref, ms1_ref):
    deg = degp_ref[0, :N] + degp_ref[1, :N] + 1.0
    dinv = lax.rsqrt(jnp.maximum(deg, 1e-12))
    dinv_ref[...] = dinv
    ms1_ref[...] = dinv[:, None] * m1_ref[...]


def _mid_body(agg_ref, dinv_ref, ms1_ref, b1_ref, w2_ref, ms2_ref):
    dinv = dinv_ref[...]
    agg = agg_ref[0, :N] + agg_ref[1, :N] + ms1_ref[...]
    h1 = jnp.maximum(dinv[:, None] * agg + b1_ref[...], 0.0)
    m2 = jnp.dot(h1, w2_ref[...], preferred_element_type=jnp.float32)
    ms2_ref[...] = dinv[:, None] * m2


def _final_body(agg_ref, dinv_ref, ms2_ref, b2_ref, batch_ref, wfc_ref,
                bfc_ref, out_ref):
    dinv = dinv_ref[...]
    agg = agg_ref[0, :N] + agg_ref[1, :N] + ms2_ref[...]
    h2 = jnp.maximum(dinv[:, None] * agg + b2_ref[...], 0.0)
    gids = lax.broadcasted_iota(jnp.int32, (N, G), 1)
    oh = (batch_ref[...][:, None] == gids).astype(jnp.float32)
    sums = lax.dot_general(oh, h2, (((0,), (0,)), ((), ())),
                           preferred_element_type=jnp.float32)
    cnt = jnp.sum(oh, axis=0)
    pooled = sums / jnp.maximum(cnt, 1.0)[:, None]
    out_ref[...] = jnp.dot(pooled, wfc_ref[...],
                           preferred_element_type=jnp.float32) + bfc_ref[...]


def kernel(x, edge_index, batch, W1, b1, W2, b2, Wfc, bfc):
    e = edge_index.shape[1]
    assert e % NW == 0
    ew = e // NW
    src = lax.optimization_barrier(edge_index[0])
    dst = lax.optimization_barrier(edge_index[1])

    degp = _make_deg_kernel(ew)(dst, jnp.zeros((NPAD,), jnp.float32))

    m1 = pl.pallas_call(
        _m1_body,
        out_shape=jax.ShapeDtypeStruct((N, 16), jnp.float32),
    )(x, W1)

    dinv, ms1 = pl.pallas_call(
        _prep1_body,
        out_shape=[jax.ShapeDtypeStruct((N,), jnp.float32),
                   jax.ShapeDtypeStruct((N, 16), jnp.float32)],
    )(m1, degp)

    agg1 = _make_agg_kernel(ew, 16)(ms1, src, dst,
                                    jnp.zeros((NPAD, 16), jnp.float32))

    ms2 = pl.pallas_call(
        _mid_body,
        out_shape=jax.ShapeDtypeStruct((N, 32), jnp.float32),
    )(agg1, dinv, ms1, b1, W2)

    agg2 = _make_agg_kernel(ew, 32)(ms2, src, dst,
                                    jnp.zeros((NPAD, 32), jnp.float32))

    out = pl.pallas_call(
        _final_body,
        out_shape=jax.ShapeDtypeStruct((G, C), jnp.float32),
    )(agg2, dinv, ms2, b2, batch.astype(jnp.int32), Wfc, bfc)
    return out


# speedup vs baseline: 1.2765x; 1.0221x over previous
import functools

import jax
import jax.numpy as jnp
from jax import lax
from jax.experimental import pallas as pl
from jax.experimental.pallas import tpu as pltpu
from jax.experimental.pallas import tpu_sc as plsc

N = 10000
F = 128
G = 16
C = 10

NC = 2
NS = 16
NW = NC * NS
CH = 512

NPAD = 10240
RPS = NPAD // NS

NBUF = 4
PREF = 2


def _chunks(ew):
    out = []
    off = 0
    while off < ew:
        ln = min(CH, ew - off)
        out.append((off, ln))
        off += ln
    return out


def _make_deg_kernel(ew):
    mesh = plsc.VectorSubcoreMesh(core_axis_name="c", subcore_axis_name="s")
    chunks = _chunks(ew)

    @functools.partial(
        pl.kernel,
        out_type=jax.ShapeDtypeStruct((NC, NPAD), jnp.float32),
        mesh=mesh,
        scratch_types=[
            pltpu.VMEM((ew,), jnp.int32),
            pltpu.VMEM((CH,), jnp.float32),
            pltpu.VMEM((RPS,), jnp.float32),
            pltpu.VMEM_SHARED((NPAD,), jnp.float32),
            pltpu.SemaphoreType.DMA,
        ],
        compiler_params=pltpu.CompilerParams(use_tc_tiling_on_sc=False),
    )
    def deg_kernel(dst_hbm, z_hbm, out_hbm, dst_v, ones_v, stage_v, deg_sh,
                   sem):
        cid = lax.axis_index("c")
        sid = lax.axis_index("s")
        wid = sid * NC + cid
        pltpu.sync_copy(dst_hbm.at[pl.ds(wid * ew, ew)], dst_v)
        def orow(i, _):
            ones_v[pl.ds(i * 16, 16)] = jnp.ones((16,), jnp.float32)
            return 0
        lax.fori_loop(0, CH // 16, orow, 0)
        pltpu.sync_copy(z_hbm.at[pl.ds(sid * RPS, RPS)],
                        deg_sh.at[pl.ds(sid * RPS, RPS)])
        plsc.subcore_barrier()

        descs = [pltpu.async_copy(ones_v.at[pl.ds(0, ln)],
                                  deg_sh.at[dst_v.at[pl.ds(off, ln)]],
                                  sem, add=True)
                 for off, ln in chunks]
        for d in descs:
            d.wait()
        plsc.subcore_barrier()
        pltpu.sync_copy(deg_sh.at[pl.ds(sid * RPS, RPS)], stage_v)
        pltpu.sync_copy(stage_v, out_hbm.at[cid, pl.ds(sid * RPS, RPS)])

    return deg_kernel


def _make_agg_kernel(ew, dout):
    mesh = plsc.VectorSubcoreMesh(core_axis_name="c", subcore_axis_name="s")
    chunks = _chunks(ew)
    kch = len(chunks)
    assert kch >= NBUF

    @functools.partial(
        pl.kernel,
        out_type=jax.ShapeDtypeStruct((NC, NPAD, dout), jnp.float32),
        mesh=mesh,
        scratch_types=(
            [pltpu.VMEM((ew,), jnp.int32),
             pltpu.VMEM((ew,), jnp.int32),
             pltpu.VMEM((RPS, dout), jnp.float32),
             pltpu.VMEM_SHARED((NPAD, dout), jnp.float32)]
            + [pltpu.VMEM((CH, dout), jnp.float32) for _ in range(NBUF)]
            + [pltpu.SemaphoreType.DMA for _ in range(2 * NBUF)]
        ),
        compiler_params=pltpu.CompilerParams(use_tc_tiling_on_sc=False),
    )
    def agg_kernel(ms_hbm, src_hbm, dst_hbm, z_hbm, out_hbm,
                   src_v, dst_v, stage_v, agg_sh, *bufs_and_sems):
        vals = bufs_and_sems[:NBUF]
        gsem = bufs_and_sems[NBUF:2 * NBUF]
        ssem = bufs_and_sems[2 * NBUF:]
        cid = lax.axis_index("c")
        sid = lax.axis_index("s")
        wid = sid * NC + cid
        pltpu.sync_copy(src_hbm.at[pl.ds(wid * ew, ew)], src_v)
        pltpu.sync_copy(dst_hbm.at[pl.ds(wid * ew, ew)], dst_v)
        pltpu.sync_copy(z_hbm.at[pl.ds(sid * RPS, RPS)],
                        agg_sh.at[pl.ds(sid * RPS, RPS)])
        plsc.subcore_barrier()

        def start_gather(j):
            off, ln = chunks[j]
            b = j % NBUF
            return pltpu.async_copy(ms_hbm.at[src_v.at[pl.ds(off, ln)]],
                                    vals[b].at[pl.ds(0, ln)], gsem[b])

        def start_scatter(j):
            off, ln = chunks[j]
            b = j % NBUF
            return pltpu.async_copy(vals[b].at[pl.ds(0, ln)],
                                    agg_sh.at[dst_v.at[pl.ds(off, ln)]],
                                    ssem[b], add=True)

        gd = [None] * kch
        sd = [None] * kch
        for j in range(PREF):
            gd[j] = start_gather(j)
        for j in range(kch):
            jp = j + PREF
            if jp < kch:
                if jp >= NBUF:
                    sd[jp - NBUF].wait()
                gd[jp] = start_gather(jp)
            gd[j].wait()
            sd[j] = start_scatter(j)
        for j in range(kch - NBUF, kch):
            sd[j].wait()
        plsc.subcore_barrier()
        pltpu.sync_copy(agg_sh.at[pl.ds(sid * RPS, RPS)], stage_v)
        pltpu.sync_copy(stage_v, out_hbm.at[cid, pl.ds(sid * RPS, RPS)])

    return agg_kernel


def _m1_body(x_ref, w1_ref, m1_ref):
    m1_ref[...] = jnp.dot(x_ref[...], w1_ref[...],
                          preferred_element_type=jnp.float32)


def _prep1_body(m1_ref, degp_ref, dinv_ref, ms1_ref):
    deg = degp_ref[0, :N] + degp_ref[1, :N] + 1.0
    dinv = lax.rsqrt(jnp.maximum(deg, 1e-12))
    dinv_ref[...] = dinv
    ms1_ref[...] = dinv[:, None] * m1_ref[...]


def _mid_body(agg_ref, dinv_ref, ms1_ref, b1_ref, w2_ref, ms2_ref):
    dinv = dinv_ref[...]
    agg = agg_ref[0, :N] + agg_ref[1, :N] + ms1_ref[...]
    h1 = jnp.maximum(dinv[:, None] * agg + b1_ref[...], 0.0)
    m2 = jnp.dot(h1, w2_ref[...], preferred_element_type=jnp.float32)
    ms2_ref[...] = dinv[:, None] * m2


def _final_body(agg_ref, dinv_ref, ms2_ref, b2_ref, batch_ref, wfc_ref,
                bfc_ref, out_ref):
    dinv = dinv_ref[...]
    agg = agg_ref[0, :N] + agg_ref[1, :N] + ms2_ref[...]
    h2 = jnp.maximum(dinv[:, None] * agg + b2_ref[...], 0.0)
    gids = lax.broadcasted_iota(jnp.int32, (N, G), 1)
    oh = (batch_ref[...][:, None] == gids).astype(jnp.float32)
    sums = lax.dot_general(oh, h2, (((0,), (0,)), ((), ())),
                           preferred_element_type=jnp.float32)
    cnt = jnp.sum(oh, axis=0)
    pooled = sums / jnp.maximum(cnt, 1.0)[:, None]
    out_ref[...] = jnp.dot(pooled, wfc_ref[...],
                           preferred_element_type=jnp.float32) + bfc_ref[...]


def kernel(x, edge_index, batch, W1, b1, W2, b2, Wfc, bfc):
    e = edge_index.shape[1]
    assert e % NW == 0
    ew = e // NW
    src = lax.optimization_barrier(edge_index[0])
    dst = lax.optimization_barrier(edge_index[1])

    degp = _make_deg_kernel(ew)(dst, jnp.zeros((NPAD,), jnp.float32))

    m1 = pl.pallas_call(
        _m1_body,
        out_shape=jax.ShapeDtypeStruct((N, 16), jnp.float32),
    )(x, W1)

    dinv, ms1 = pl.pallas_call(
        _prep1_body,
        out_shape=[jax.ShapeDtypeStruct((N,), jnp.float32),
                   jax.ShapeDtypeStruct((N, 16), jnp.float32)],
    )(m1, degp)

    agg1 = _make_agg_kernel(ew, 16)(ms1, src, dst,
                                    jnp.zeros((NPAD, 16), jnp.float32))

    ms2 = pl.pallas_call(
        _mid_body,
        out_shape=jax.ShapeDtypeStruct((N, 32), jnp.float32),
    )(agg1, dinv, ms1, b1, W2)

    agg2 = _make_agg_kernel(ew, 32)(ms2, src, dst,
                                    jnp.zeros((NPAD, 32), jnp.float32))

    out = pl.pallas_call(
        _final_body,
        out_shape=jax.ShapeDtypeStruct((G, C), jnp.float32),
    )(agg2, dinv, ms2, b2, batch.astype(jnp.int32), Wfc, bfc)
    return out
